# Initial kernel scaffold; baseline (speedup 1.0000x reference)
import functools

import jax
import jax.numpy as jnp
from jax import lax
from jax.experimental import pallas as pl
from jax.experimental.pallas import tpu as pltpu
from jax.experimental.pallas import tpu_sc as plsc

_B = 4
_N = 5000
_G = 32
_NP = _N + _G
_TPB = 8
_RPT = _NP // _TPB
_RPAD = 640
_NCHUNK = _RPAD // 16
_POS_CAP = 64
_NEG_CAP = 256
_POS_IOU = 0.7
_NEG_IOU = 0.3


def _sc_body(props_hbm, gt_hbm, cls_hbm, rois_hbm, gtb_hbm, lab_hbm,
             props_v, gt_v, garea_v, cls_v,
             px1, py1, px2, py2, pg, nx1, ny1, nx2, ny2, ng, cnt_v,
             apos_f, apos_g, aneg_f, aneg_g, acnt,
             rois_v, gtb_v, lab_v,
             sp_pos_f, sp_pos_g, sp_neg_f, sp_neg_g, sp_cnt):
    c = lax.axis_index("c")
    s = lax.axis_index("s")
    bl = s // _TPB
    sub = s % _TPB
    b = c * 2 + bl

    iota16 = lax.broadcasted_iota(jnp.int32, (16,), 0)

    pltpu.sync_copy(props_hbm.at[b, sub], props_v)
    pltpu.sync_copy(gt_hbm.at[b], gt_v)
    pltpu.sync_copy(cls_hbm.at[b], cls_v)

    for half in range(2):
        gx1 = gt_v[0, pl.ds(half * 16, 16)]
        gy1 = gt_v[1, pl.ds(half * 16, 16)]
        gx2 = gt_v[2, pl.ds(half * 16, 16)]
        gy2 = gt_v[3, pl.ds(half * 16, 16)]
        garea_v[pl.ds(half * 16, 16)] = (gx2 - gx1) * (gy2 - gy1)

    def chunk_body(chunk, carry):
        pos_cnt, neg_cnt = carry
        base = chunk * 16
        ax1 = props_v[0, pl.ds(base, 16)]
        ay1 = props_v[1, pl.ds(base, 16)]
        ax2 = props_v[2, pl.ds(base, 16)]
        ay2 = props_v[3, pl.ds(base, 16)]
        area_a = (ax2 - ax1) * (ay2 - ay1)

        miou = jnp.full((16,), -1.0, jnp.float32)
        mg = jnp.zeros((16,), jnp.int32)
        for j in range(_G):
            gx1 = gt_v[0, j]
            gy1 = gt_v[1, j]
            gx2 = gt_v[2, j]
            gy2 = gt_v[3, j]
            ga = garea_v[j]
            w = jnp.maximum(jnp.minimum(ax2, gx2) - jnp.maximum(ax1, gx1), 0.0)
            h = jnp.maximum(jnp.minimum(ay2, gy2) - jnp.maximum(ay1, gy1), 0.0)
            inter = w * h
            iou = inter / (area_a + ga - inter)
            upd = iou > miou
            miou = jnp.where(upd, iou, miou)
            mg = jnp.where(upd, jnp.int32(j), mg)

        lanev = (base + iota16) < _RPT
        pos_m = jnp.logical_and(miou >= _POS_IOU, lanev)
        neg_m = jnp.logical_and(miou < _NEG_IOU, lanev)

        can_p = pos_cnt < _POS_CAP
        pm = jnp.logical_and(pos_m, can_p)
        plsc.store_compressed(px1.at[pl.ds(pos_cnt, 16)], ax1, mask=pm)
        plsc.store_compressed(py1.at[pl.ds(pos_cnt, 16)], ay1, mask=pm)
        plsc.store_compressed(px2.at[pl.ds(pos_cnt, 16)], ax2, mask=pm)
        plsc.store_compressed(py2.at[pl.ds(pos_cnt, 16)], ay2, mask=pm)
        plsc.store_compressed(pg.at[pl.ds(pos_cnt, 16)], mg, mask=pm)
        pos_cnt = jnp.where(can_p,
                            pos_cnt + jnp.sum(pos_m.astype(jnp.int32)),
                            pos_cnt)

        can_n = neg_cnt < _NEG_CAP
        nm = jnp.logical_and(neg_m, can_n)
        plsc.store_compressed(nx1.at[pl.ds(neg_cnt, 16)], ax1, mask=nm)
        plsc.store_compressed(ny1.at[pl.ds(neg_cnt, 16)], ay1, mask=nm)
        plsc.store_compressed(nx2.at[pl.ds(neg_cnt, 16)], ax2, mask=nm)
        plsc.store_compressed(ny2.at[pl.ds(neg_cnt, 16)], ay2, mask=nm)
        plsc.store_compressed(ng.at[pl.ds(neg_cnt, 16)], mg, mask=nm)
        neg_cnt = jnp.where(can_n,
                            neg_cnt + jnp.sum(neg_m.astype(jnp.int32)),
                            neg_cnt)
        return pos_cnt, neg_cnt

    pos_cnt, neg_cnt = lax.fori_loop(
        0, _NCHUNK, chunk_body,
        (jnp.int32(0), jnp.int32(0)))

    posn = jnp.minimum(pos_cnt, _POS_CAP)
    negn = jnp.minimum(neg_cnt, _NEG_CAP)
    cnt_v[...] = jnp.where(iota16 == 0, posn,
                           jnp.where(iota16 == 1, negn, 0))

    pltpu.sync_copy(px1.at[pl.ds(0, _POS_CAP)], sp_pos_f.at[bl, 0, sub])
    pltpu.sync_copy(py1.at[pl.ds(0, _POS_CAP)], sp_pos_f.at[bl, 1, sub])
    pltpu.sync_copy(px2.at[pl.ds(0, _POS_CAP)], sp_pos_f.at[bl, 2, sub])
    pltpu.sync_copy(py2.at[pl.ds(0, _POS_CAP)], sp_pos_f.at[bl, 3, sub])
    pltpu.sync_copy(pg.at[pl.ds(0, _POS_CAP)], sp_pos_g.at[bl, sub])
    pltpu.sync_copy(nx1.at[pl.ds(0, _NEG_CAP)], sp_neg_f.at[bl, 0, sub])
    pltpu.sync_copy(ny1.at[pl.ds(0, _NEG_CAP)], sp_neg_f.at[bl, 1, sub])
    pltpu.sync_copy(nx2.at[pl.ds(0, _NEG_CAP)], sp_neg_f.at[bl, 2, sub])
    pltpu.sync_copy(ny2.at[pl.ds(0, _NEG_CAP)], sp_neg_f.at[bl, 3, sub])
    pltpu.sync_copy(ng.at[pl.ds(0, _NEG_CAP)], sp_neg_g.at[bl, sub])
    pltpu.sync_copy(cnt_v, sp_cnt.at[bl, sub])

    plsc.subcore_barrier()

    @pl.when(sub == 0)
    def _assemble():
        pltpu.sync_copy(sp_pos_f.at[bl], apos_f)
        pltpu.sync_copy(sp_pos_g.at[bl], apos_g)
        pltpu.sync_copy(sp_neg_f.at[bl], aneg_f)
        pltpu.sync_copy(sp_neg_g.at[bl], aneg_g)
        pltpu.sync_copy(sp_cnt.at[bl], acnt)

        pp = [jnp.int32(0)]
        nn = [jnp.int32(0)]
        for t in range(_TPB):
            pp.append(pp[-1] + acnt[t, 0])
            nn.append(nn[-1] + acnt[t, 1])
        num_pos = jnp.minimum(pp[_TPB], _POS_CAP)
        num_neg = jnp.minimum(nn[_TPB], _NEG_CAP - num_pos)
        tot = num_pos + num_neg

        for chunk in range(_NEG_CAP // 16):
            sv = chunk * 16 + iota16
            is_pos = sv < num_pos
            valid = sv < tot

            tp = jnp.zeros((16,), jnp.int32)
            pv = jnp.zeros((16,), jnp.int32)
            sn = sv - num_pos
            tn = jnp.zeros((16,), jnp.int32)
            nv = jnp.zeros((16,), jnp.int32)
            for t in range(1, _TPB):
                tp = tp + (sv >= pp[t]).astype(jnp.int32)
                tn = tn + (sn >= nn[t]).astype(jnp.int32)
            for t in range(1, _TPB):
                pv = jnp.where(tp == t, pp[t], pv)
                nv = jnp.where(tn == t, nn[t], nv)
            lp = jnp.clip(sv - pv, 0, _POS_CAP - 1)
            ln = jnp.clip(sn - nv, 0, _NEG_CAP - 1)

            gp = plsc.load_gather(apos_g, [tp, lp])
            gn = plsc.load_gather(aneg_g, [tn, ln])
            g = jnp.clip(jnp.where(is_pos, gp, gn), 0, _G - 1)

            lab = jnp.where(is_pos, plsc.load_gather(cls_v, [g]), 0)
            lab_v[pl.ds(chunk * 16, 16)] = lab

            for f in range(4):
                pf = plsc.load_gather(apos_f.at[f], [tp, lp])
                nf = plsc.load_gather(aneg_f.at[f], [tn, ln])
                rois_v[f, pl.ds(chunk * 16, 16)] = jnp.where(
                    valid, jnp.where(is_pos, pf, nf), 0.0)
                gf = plsc.load_gather(
                    gt_v, [jnp.full((16,), f, jnp.int32), g])
                gtb_v[f, pl.ds(chunk * 16, 16)] = jnp.where(valid, gf, 0.0)

        pltpu.sync_copy(rois_v, rois_hbm.at[b])
        pltpu.sync_copy(gtb_v, gtb_hbm.at[b])
        pltpu.sync_copy(lab_v, lab_hbm.at[b])


_sc_kernel = functools.partial(
    pl.kernel,
    out_type=[
        jax.ShapeDtypeStruct((_B, 4, _NEG_CAP), jnp.float32),
        jax.ShapeDtypeStruct((_B, 4, _NEG_CAP), jnp.float32),
        jax.ShapeDtypeStruct((_B, _NEG_CAP), jnp.int32),
    ],
    mesh=plsc.VectorSubcoreMesh(core_axis_name="c", subcore_axis_name="s"),
    scratch_types=[
        pltpu.VMEM((4, _RPAD), jnp.float32),
        pltpu.VMEM((4, _G), jnp.float32),
        pltpu.VMEM((_G,), jnp.float32),
        pltpu.VMEM((_G,), jnp.int32),
        pltpu.VMEM((_POS_CAP + 16,), jnp.float32),
        pltpu.VMEM((_POS_CAP + 16,), jnp.float32),
        pltpu.VMEM((_POS_CAP + 16,), jnp.float32),
        pltpu.VMEM((_POS_CAP + 16,), jnp.float32),
        pltpu.VMEM((_POS_CAP + 16,), jnp.int32),
        pltpu.VMEM((_NEG_CAP + 16,), jnp.float32),
        pltpu.VMEM((_NEG_CAP + 16,), jnp.float32),
        pltpu.VMEM((_NEG_CAP + 16,), jnp.float32),
        pltpu.VMEM((_NEG_CAP + 16,), jnp.float32),
        pltpu.VMEM((_NEG_CAP + 16,), jnp.int32),
        pltpu.VMEM((16,), jnp.int32),
        pltpu.VMEM((4, _TPB, _POS_CAP), jnp.float32),
        pltpu.VMEM((_TPB, _POS_CAP), jnp.int32),
        pltpu.VMEM((4, _TPB, _NEG_CAP), jnp.float32),
        pltpu.VMEM((_TPB, _NEG_CAP), jnp.int32),
        pltpu.VMEM((_TPB, 16), jnp.int32),
        pltpu.VMEM((4, _NEG_CAP), jnp.float32),
        pltpu.VMEM((4, _NEG_CAP), jnp.float32),
        pltpu.VMEM((_NEG_CAP,), jnp.int32),
        pltpu.VMEM_SHARED((2, 4, _TPB, _POS_CAP), jnp.float32),
        pltpu.VMEM_SHARED((2, _TPB, _POS_CAP), jnp.int32),
        pltpu.VMEM_SHARED((2, 4, _TPB, _NEG_CAP), jnp.float32),
        pltpu.VMEM_SHARED((2, _TPB, _NEG_CAP), jnp.int32),
        pltpu.VMEM_SHARED((2, _TPB, 16), jnp.int32),
    ],
)(_sc_body)


def _tc_body(rois_ref, gtb_ref, out_rois_ref, out_off_ref):
    x1 = rois_ref[:, 0, :]
    y1 = rois_ref[:, 1, :]
    x2 = rois_ref[:, 2, :]
    y2 = rois_ref[:, 3, :]
    gx1 = gtb_ref[:, 0, :]
    gy1 = gtb_ref[:, 1, :]
    gx2 = gtb_ref[:, 2, :]
    gy2 = gtb_ref[:, 3, :]

    cx = (x1 + x2) * 0.5
    cy = (y1 + y2) * 0.5
    w = x2 - x1
    h = y2 - y1
    gcx = (gx1 + gx2) * 0.5
    gcy = (gy1 + gy2) * 0.5
    gw = gx2 - gx1
    gh = gy2 - gy1

    out_off_ref[:, 0, :] = 10.0 * (gcx - cx) / w
    out_off_ref[:, 1, :] = 10.0 * (gcy - cy) / h
    out_off_ref[:, 2, :] = 5.0 * jnp.log(gw / w)
    out_off_ref[:, 3, :] = 5.0 * jnp.log(gh / h)

    out_rois_ref[:, 0, :] = cx - w * 0.5
    out_rois_ref[:, 1, :] = cy - h * 0.5
    out_rois_ref[:, 2, :] = cx + w * 0.5
    out_rois_ref[:, 3, :] = cy + h * 0.5


_tc_epilogue = pl.pallas_call(
    _tc_body,
    out_shape=[
        jax.ShapeDtypeStruct((_B, 4, _NEG_CAP), jnp.float32),
        jax.ShapeDtypeStruct((_B, 4, _NEG_CAP), jnp.float32),
    ],
)


def kernel(all_proposals, all_gt_bboxes, all_gt_orig_classes):
    props = jnp.concatenate([all_proposals, all_gt_bboxes], axis=1)
    props_t = props.reshape(_B, _TPB, _RPT, 4)
    props_t = jnp.pad(props_t, ((0, 0), (0, 0), (0, _RPAD - _RPT), (0, 0)))
    props_t = props_t.transpose(0, 1, 3, 2)
    gt_t = all_gt_bboxes.transpose(0, 2, 1)
    cls = all_gt_orig_classes.astype(jnp.int32)

    rois_t, gtb_t, labels = _sc_kernel(props_t, gt_t, cls)
    rois_out_t, off_t = _tc_epilogue(rois_t, gtb_t)

    rois_out = rois_out_t.transpose(0, 2, 1)
    gt_offsets = off_t.transpose(0, 2, 1)
    return rois_out, labels, gt_offsets


# speedup vs baseline: 5.3239x; 5.3239x over previous
import functools

import jax
import jax.numpy as jnp
from jax import lax
from jax.experimental import pallas as pl
from jax.experimental.pallas import tpu as pltpu
from jax.experimental.pallas import tpu_sc as plsc

_B = 4
_N = 5000
_G = 32
_NP = _N + _G
_TPB = 8
_RPT = _NP // _TPB
_RPAD = 640
_NCHUNK = _RPAD // 16
_POS_CAP = 64
_NEG_CAP = 256
_POS_IOU = 0.7
_NEG_IOU = 0.3


def _sc_body(props_hbm, gt_hbm, cls_hbm, rois_hbm, gtb_hbm, lab_hbm,
             praw_v, gt_v, cls_v,
             px1, py1, px2, py2, pg, nx1, ny1, nx2, ny2, ng,
             apos_f, apos_g, aneg_f, aneg_g,
             rois_v, gtb_v, lab_v,
             sp_pos_f, sp_pos_g, sp_neg_f, sp_neg_g):
    c = lax.axis_index("c")
    s = lax.axis_index("s")
    bl = s // _TPB
    sub = s % _TPB
    b = c * 2 + bl

    iota16 = lax.broadcasted_iota(jnp.int32, (16,), 0)
    fcol = [jnp.full((16,), f, jnp.int32) for f in range(4)]

    start8 = sub * _RPT // 8 * 8
    shift = sub * _RPT - start8
    _L7 = (_TPB - 1) * _RPT // 8 * 8
    _N7 = _N - _L7

    @pl.when(sub < _TPB - 1)
    def _stage_mid():
        pltpu.sync_copy(props_hbm.at[b, pl.ds(start8, _RPAD)],
                        praw_v.at[pl.ds(0, _RPAD)])

    @pl.when(sub == _TPB - 1)
    def _stage_last():
        pltpu.sync_copy(props_hbm.at[b, pl.ds(_L7, _N7)],
                        praw_v.at[pl.ds(0, _N7)])
        pltpu.sync_copy(gt_hbm.at[b], praw_v.at[pl.ds(_N7, _G)])

    pltpu.sync_copy(gt_hbm.at[b], gt_v)
    pltpu.sync_copy(cls_hbm.at[b], cls_v)

    gx1v = [plsc.load_gather(gt_v, [iota16 + h * 16, fcol[0]]) for h in range(2)]
    gy1v = [plsc.load_gather(gt_v, [iota16 + h * 16, fcol[1]]) for h in range(2)]
    gx2v = [plsc.load_gather(gt_v, [iota16 + h * 16, fcol[2]]) for h in range(2)]
    gy2v = [plsc.load_gather(gt_v, [iota16 + h * 16, fcol[3]]) for h in range(2)]
    gav = [(gx2v[h] - gx1v[h]) * (gy2v[h] - gy1v[h]) for h in range(2)]

    def chunk_body(chunk, carry):
        pos_cnt, neg_cnt = carry
        base = chunk * 16
        ridx = shift + base + iota16
        ax1 = plsc.load_gather(praw_v, [ridx, fcol[0]])
        ay1 = plsc.load_gather(praw_v, [ridx, fcol[1]])
        ax2 = plsc.load_gather(praw_v, [ridx, fcol[2]])
        ay2 = plsc.load_gather(praw_v, [ridx, fcol[3]])
        area_a = (ax2 - ax1) * (ay2 - ay1)

        miou = jnp.full((16,), -1.0, jnp.float32)
        mg = jnp.zeros((16,), jnp.int32)
        for j in range(_G):
            h, l = divmod(j, 16)
            gx1 = gx1v[h][l]
            gy1 = gy1v[h][l]
            gx2 = gx2v[h][l]
            gy2 = gy2v[h][l]
            ga = gav[h][l]
            w = jnp.maximum(jnp.minimum(ax2, gx2) - jnp.maximum(ax1, gx1), 0.0)
            h = jnp.maximum(jnp.minimum(ay2, gy2) - jnp.maximum(ay1, gy1), 0.0)
            inter = w * h
            iou = inter / (area_a + ga - inter)
            upd = iou > miou
            miou = jnp.where(upd, iou, miou)
            mg = jnp.where(upd, jnp.int32(j), mg)

        lanev = (base + iota16) < _RPT
        pos_m = jnp.logical_and(miou >= _POS_IOU, lanev)
        neg_m = jnp.logical_and(miou < _NEG_IOU, lanev)

        can_p = pos_cnt < _POS_CAP
        pm = jnp.logical_and(pos_m, can_p)
        pfx = plsc.cumsum(pos_m.astype(jnp.int32))
        pidx = jnp.where(pm, pos_cnt + pfx - 1, jnp.int32(_POS_CAP + 15))
        plsc.store_scatter(px1, [pidx], ax1)
        plsc.store_scatter(py1, [pidx], ay1)
        plsc.store_scatter(px2, [pidx], ax2)
        plsc.store_scatter(py2, [pidx], ay2)
        plsc.store_scatter(pg, [pidx], mg)
        pos_cnt = jnp.where(can_p, pos_cnt + pfx[15], pos_cnt)

        can_n = neg_cnt < _NEG_CAP
        nm = jnp.logical_and(neg_m, can_n)
        nfx = plsc.cumsum(neg_m.astype(jnp.int32))
        nidx = jnp.where(nm, neg_cnt + nfx - 1, jnp.int32(_NEG_CAP + 15))
        plsc.store_scatter(nx1, [nidx], ax1)
        plsc.store_scatter(ny1, [nidx], ay1)
        plsc.store_scatter(nx2, [nidx], ax2)
        plsc.store_scatter(ny2, [nidx], ay2)
        plsc.store_scatter(ng, [nidx], mg)
        neg_cnt = jnp.where(can_n, neg_cnt + nfx[15], neg_cnt)
        return pos_cnt, neg_cnt

    pos_cnt, neg_cnt = lax.fori_loop(
        0, _NCHUNK, chunk_body,
        (jnp.int32(0), jnp.int32(0)))

    posn = jnp.minimum(pos_cnt, _POS_CAP)
    negn = jnp.minimum(neg_cnt, _NEG_CAP)
    pg[pl.ds(_POS_CAP, 16)] = jnp.where(iota16 == 0, posn,
                                        jnp.where(iota16 == 1, negn, 0))

    prow = bl * (4 * _TPB) + sub
    pltpu.sync_copy(px1.at[pl.ds(0, _POS_CAP)], sp_pos_f.at[prow + 0 * _TPB])
    pltpu.sync_copy(py1.at[pl.ds(0, _POS_CAP)], sp_pos_f.at[prow + 1 * _TPB])
    pltpu.sync_copy(px2.at[pl.ds(0, _POS_CAP)], sp_pos_f.at[prow + 2 * _TPB])
    pltpu.sync_copy(py2.at[pl.ds(0, _POS_CAP)], sp_pos_f.at[prow + 3 * _TPB])
    pltpu.sync_copy(pg, sp_pos_g.at[bl * _TPB + sub])
    pltpu.sync_copy(nx1.at[pl.ds(0, _NEG_CAP)], sp_neg_f.at[prow + 0 * _TPB])
    pltpu.sync_copy(ny1.at[pl.ds(0, _NEG_CAP)], sp_neg_f.at[prow + 1 * _TPB])
    pltpu.sync_copy(nx2.at[pl.ds(0, _NEG_CAP)], sp_neg_f.at[prow + 2 * _TPB])
    pltpu.sync_copy(ny2.at[pl.ds(0, _NEG_CAP)], sp_neg_f.at[prow + 3 * _TPB])
    pltpu.sync_copy(ng.at[pl.ds(0, _NEG_CAP)], sp_neg_g.at[bl * _TPB + sub])

    plsc.subcore_barrier()

    @pl.when(sub == 0)
    def _assemble():
        pltpu.sync_copy(sp_pos_f.at[pl.ds(bl * (4 * _TPB), 4 * _TPB)], apos_f)
        pltpu.sync_copy(sp_pos_g.at[pl.ds(bl * _TPB, _TPB)], apos_g)
        pltpu.sync_copy(sp_neg_f.at[pl.ds(bl * (4 * _TPB), 4 * _TPB)], aneg_f)
        pltpu.sync_copy(sp_neg_g.at[pl.ds(bl * _TPB, _TPB)], aneg_g)

        pp = [jnp.int32(0)]
        nn = [jnp.int32(0)]
        for t in range(_TPB):
            crow = apos_g[t, pl.ds(_POS_CAP, 16)]
            pp.append(pp[-1] + crow[0])
            nn.append(nn[-1] + crow[1])
        num_pos = jnp.minimum(pp[_TPB], _POS_CAP)
        num_neg = jnp.minimum(nn[_TPB], _NEG_CAP - num_pos)
        tot = num_pos + num_neg

        for chunk in range(_NEG_CAP // 16):
            sv = chunk * 16 + iota16
            is_pos = sv < num_pos
            valid = sv < tot

            tp = jnp.zeros((16,), jnp.int32)
            pv = jnp.zeros((16,), jnp.int32)
            sn = sv - num_pos
            tn = jnp.zeros((16,), jnp.int32)
            nv = jnp.zeros((16,), jnp.int32)
            for t in range(1, _TPB):
                tp = tp + (sv >= pp[t]).astype(jnp.int32)
                tn = tn + (sn >= nn[t]).astype(jnp.int32)
            for t in range(1, _TPB):
                pv = jnp.where(tp == t, pp[t], pv)
                nv = jnp.where(tn == t, nn[t], nv)
            lp = jnp.clip(sv - pv, 0, _POS_CAP - 1)
            ln = jnp.clip(sn - nv, 0, _NEG_CAP - 1)

            gp = plsc.load_gather(apos_g, [tp, lp])
            gn = plsc.load_gather(aneg_g, [tn, ln])
            g = jnp.clip(jnp.where(is_pos, gp, gn), 0, _G - 1)

            lab = jnp.where(is_pos, plsc.load_gather(cls_v, [g]), 0)
            lab_v[pl.ds(chunk * 16, 16)] = lab

            for f in range(4):
                pf = plsc.load_gather(apos_f, [tp + f * _TPB, lp])
                nf = plsc.load_gather(aneg_f, [tn + f * _TPB, ln])
                rois_v[f, pl.ds(chunk * 16, 16)] = jnp.where(
                    valid, jnp.where(is_pos, pf, nf), 0.0)
                gf = plsc.load_gather(gt_v, [g, fcol[f]])
                gtb_v[f, pl.ds(chunk * 16, 16)] = jnp.where(valid, gf, 0.0)

        pltpu.sync_copy(rois_v, rois_hbm.at[b])
        pltpu.sync_copy(gtb_v, gtb_hbm.at[b])
        pltpu.sync_copy(lab_v, lab_hbm.at[b])


_sc_kernel = functools.partial(
    pl.kernel,
    compiler_params=pltpu.CompilerParams(needs_layout_passes=False),
    out_type=[
        jax.ShapeDtypeStruct((_B, 4, _NEG_CAP), jnp.float32),
        jax.ShapeDtypeStruct((_B, 4, _NEG_CAP), jnp.float32),
        jax.ShapeDtypeStruct((_B, _NEG_CAP), jnp.int32),
    ],
    mesh=plsc.VectorSubcoreMesh(core_axis_name="c", subcore_axis_name="s",
                                num_cores=2, num_subcores=16),
    scratch_types=[
        pltpu.VMEM((_RPAD + 16, 4), jnp.float32),
        pltpu.VMEM((_G, 4), jnp.float32),
        pltpu.VMEM((_G,), jnp.int32),
        pltpu.VMEM((_POS_CAP + 16,), jnp.float32),
        pltpu.VMEM((_POS_CAP + 16,), jnp.float32),
        pltpu.VMEM((_POS_CAP + 16,), jnp.float32),
        pltpu.VMEM((_POS_CAP + 16,), jnp.float32),
        pltpu.VMEM((_POS_CAP + 16,), jnp.int32),
        pltpu.VMEM((_NEG_CAP + 16,), jnp.float32),
        pltpu.VMEM((_NEG_CAP + 16,), jnp.float32),
        pltpu.VMEM((_NEG_CAP + 16,), jnp.float32),
        pltpu.VMEM((_NEG_CAP + 16,), jnp.float32),
        pltpu.VMEM((_NEG_CAP + 16,), jnp.int32),
        pltpu.VMEM((4 * _TPB, _POS_CAP), jnp.float32),
        pltpu.VMEM((_TPB, _POS_CAP + 16), jnp.int32),
        pltpu.VMEM((4 * _TPB, _NEG_CAP), jnp.float32),
        pltpu.VMEM((_TPB, _NEG_CAP), jnp.int32),
        pltpu.VMEM((4, _NEG_CAP), jnp.float32),
        pltpu.VMEM((4, _NEG_CAP), jnp.float32),
        pltpu.VMEM((_NEG_CAP,), jnp.int32),
        pltpu.VMEM_SHARED((2 * 2 * 4 * _TPB, _POS_CAP), jnp.float32),
        pltpu.VMEM_SHARED((2 * 2 * _TPB, _POS_CAP + 16), jnp.int32),
        pltpu.VMEM_SHARED((2 * 2 * 4 * _TPB, _NEG_CAP), jnp.float32),
        pltpu.VMEM_SHARED((2 * 2 * _TPB, _NEG_CAP), jnp.int32),
    ],
)(_sc_body)


def _tc_body(rois_ref, gtb_ref, out_rois_ref, out_off_ref):
    x1 = rois_ref[:, 0, :]
    y1 = rois_ref[:, 1, :]
    x2 = rois_ref[:, 2, :]
    y2 = rois_ref[:, 3, :]
    gx1 = gtb_ref[:, 0, :]
    gy1 = gtb_ref[:, 1, :]
    gx2 = gtb_ref[:, 2, :]
    gy2 = gtb_ref[:, 3, :]

    cx = (x1 + x2) * 0.5
    cy = (y1 + y2) * 0.5
    w = x2 - x1
    h = y2 - y1
    gcx = (gx1 + gx2) * 0.5
    gcy = (gy1 + gy2) * 0.5
    gw = gx2 - gx1
    gh = gy2 - gy1

    out_off_ref[:, 0, :] = 10.0 * (gcx - cx) / w
    out_off_ref[:, 1, :] = 10.0 * (gcy - cy) / h
    out_off_ref[:, 2, :] = 5.0 * jnp.log(gw / w)
    out_off_ref[:, 3, :] = 5.0 * jnp.log(gh / h)

    out_rois_ref[:, 0, :] = cx - w * 0.5
    out_rois_ref[:, 1, :] = cy - h * 0.5
    out_rois_ref[:, 2, :] = cx + w * 0.5
    out_rois_ref[:, 3, :] = cy + h * 0.5


_tc_epilogue = pl.pallas_call(
    _tc_body,
    out_shape=[
        jax.ShapeDtypeStruct((_B, 4, _NEG_CAP), jnp.float32),
        jax.ShapeDtypeStruct((_B, 4, _NEG_CAP), jnp.float32),
    ],
)


def kernel(all_proposals, all_gt_bboxes, all_gt_orig_classes):
    cls = all_gt_orig_classes.astype(jnp.int32)

    rois_t, gtb_t, labels = _sc_kernel(all_proposals, all_gt_bboxes, cls)
    rois_out_t, off_t = _tc_epilogue(rois_t, gtb_t)

    rois_out = rois_out_t.transpose(0, 2, 1)
    gt_offsets = off_t.transpose(0, 2, 1)
    return rois_out, labels, gt_offsets


# speedup vs baseline: 6.6149x; 1.2425x over previous
import functools

import jax
import jax.numpy as jnp
from jax import lax
from jax.experimental import pallas as pl
from jax.experimental.pallas import tpu as pltpu
from jax.experimental.pallas import tpu_sc as plsc

_B = 4
_N = 5000
_G = 32
_NP = _N + _G
_TPB = 8
_RPT = _NP // _TPB
_RPAD = 640
_NCHUNK = _RPAD // 16
_POS_CAP = 64
_NEG_CAP = 256
_POS_IOU = 0.7
_NEG_IOU = 0.3


def _sc_body(props_hbm, gt_hbm, cls_hbm, rois_hbm, gtb_hbm, lab_hbm,
             props_v, gt_v, cls_v,
             px1, py1, px2, py2, pg, nx1, ny1, nx2, ny2, ng,
             apos_f, apos_g, aneg_f, aneg_g,
             rois_v, gtb_v, lab_v,
             sp_pos_f, sp_pos_g, sp_neg_f, sp_neg_g):
    c = lax.axis_index("c")
    s = lax.axis_index("s")
    bl = s // _TPB
    sub = s % _TPB
    b = c * 2 + bl

    iota16 = lax.broadcasted_iota(jnp.int32, (16,), 0)

    pltpu.sync_copy(props_hbm.at[b, sub], props_v)
    pltpu.sync_copy(gt_hbm.at[b], gt_v)
    pltpu.sync_copy(cls_hbm.at[b], cls_v)

    gx1v = [gt_v[0, pl.ds(h * 16, 16)] for h in range(2)]
    gy1v = [gt_v[1, pl.ds(h * 16, 16)] for h in range(2)]
    gx2v = [gt_v[2, pl.ds(h * 16, 16)] for h in range(2)]
    gy2v = [gt_v[3, pl.ds(h * 16, 16)] for h in range(2)]
    gav = [(gx2v[h] - gx1v[h]) * (gy2v[h] - gy1v[h]) for h in range(2)]

    def chunk_body(chunk, carry):
        pos_cnt, neg_cnt = carry
        base = chunk * 16
        ax1 = props_v[0, pl.ds(base, 16)]
        ay1 = props_v[1, pl.ds(base, 16)]
        ax2 = props_v[2, pl.ds(base, 16)]
        ay2 = props_v[3, pl.ds(base, 16)]
        area_a = (ax2 - ax1) * (ay2 - ay1)

        miou = jnp.full((16,), -1.0, jnp.float32)
        mg = jnp.zeros((16,), jnp.int32)
        for j in range(_G):
            h, l = divmod(j, 16)
            gx1 = gx1v[h][l]
            gy1 = gy1v[h][l]
            gx2 = gx2v[h][l]
            gy2 = gy2v[h][l]
            ga = gav[h][l]
            w = jnp.maximum(jnp.minimum(ax2, gx2) - jnp.maximum(ax1, gx1), 0.0)
            h = jnp.maximum(jnp.minimum(ay2, gy2) - jnp.maximum(ay1, gy1), 0.0)
            inter = w * h
            iou = inter / (area_a + ga - inter)
            upd = iou > miou
            miou = jnp.where(upd, iou, miou)
            mg = jnp.where(upd, jnp.int32(j), mg)

        lanev = (base + iota16) < _RPT
        pos_m = jnp.logical_and(miou >= _POS_IOU, lanev)
        neg_m = jnp.logical_and(miou < _NEG_IOU, lanev)

        can_p = pos_cnt < _POS_CAP
        pm = jnp.logical_and(pos_m, can_p)
        pfx = plsc.cumsum(pos_m.astype(jnp.int32))
        pidx = jnp.where(pm, pos_cnt + pfx - 1, jnp.int32(_POS_CAP + 15))
        plsc.store_scatter(px1, [pidx], ax1)
        plsc.store_scatter(py1, [pidx], ay1)
        plsc.store_scatter(px2, [pidx], ax2)
        plsc.store_scatter(py2, [pidx], ay2)
        plsc.store_scatter(pg, [pidx], mg)
        pos_cnt = jnp.where(can_p, pos_cnt + pfx[15], pos_cnt)

        can_n = neg_cnt < _NEG_CAP
        nm = jnp.logical_and(neg_m, can_n)
        nfx = plsc.cumsum(neg_m.astype(jnp.int32))
        nidx = jnp.where(nm, neg_cnt + nfx - 1, jnp.int32(_NEG_CAP + 15))
        plsc.store_scatter(nx1, [nidx], ax1)
        plsc.store_scatter(ny1, [nidx], ay1)
        plsc.store_scatter(nx2, [nidx], ax2)
        plsc.store_scatter(ny2, [nidx], ay2)
        plsc.store_scatter(ng, [nidx], mg)
        neg_cnt = jnp.where(can_n, neg_cnt + nfx[15], neg_cnt)
        return pos_cnt, neg_cnt

    pos_cnt, neg_cnt = lax.fori_loop(
        0, _NCHUNK, chunk_body,
        (jnp.int32(0), jnp.int32(0)))

    posn = jnp.minimum(pos_cnt, _POS_CAP)
    negn = jnp.minimum(neg_cnt, _NEG_CAP)
    pg[pl.ds(_POS_CAP, 16)] = jnp.where(iota16 == 0, posn,
                                        jnp.where(iota16 == 1, negn, 0))

    prow = bl * (4 * _TPB) + sub
    pltpu.sync_copy(px1.at[pl.ds(0, _POS_CAP)], sp_pos_f.at[prow + 0 * _TPB])
    pltpu.sync_copy(py1.at[pl.ds(0, _POS_CAP)], sp_pos_f.at[prow + 1 * _TPB])
    pltpu.sync_copy(px2.at[pl.ds(0, _POS_CAP)], sp_pos_f.at[prow + 2 * _TPB])
    pltpu.sync_copy(py2.at[pl.ds(0, _POS_CAP)], sp_pos_f.at[prow + 3 * _TPB])
    pltpu.sync_copy(pg, sp_pos_g.at[bl * _TPB + sub])
    pltpu.sync_copy(nx1.at[pl.ds(0, _NEG_CAP)], sp_neg_f.at[prow + 0 * _TPB])
    pltpu.sync_copy(ny1.at[pl.ds(0, _NEG_CAP)], sp_neg_f.at[prow + 1 * _TPB])
    pltpu.sync_copy(nx2.at[pl.ds(0, _NEG_CAP)], sp_neg_f.at[prow + 2 * _TPB])
    pltpu.sync_copy(ny2.at[pl.ds(0, _NEG_CAP)], sp_neg_f.at[prow + 3 * _TPB])
    pltpu.sync_copy(ng.at[pl.ds(0, _NEG_CAP)], sp_neg_g.at[bl * _TPB + sub])

    plsc.subcore_barrier()

    @pl.when(sub == 0)
    def _assemble():
        pltpu.sync_copy(sp_pos_f.at[pl.ds(bl * (4 * _TPB), 4 * _TPB)], apos_f)
        pltpu.sync_copy(sp_pos_g.at[pl.ds(bl * _TPB, _TPB)], apos_g)
        pltpu.sync_copy(sp_neg_f.at[pl.ds(bl * (4 * _TPB), 4 * _TPB)], aneg_f)
        pltpu.sync_copy(sp_neg_g.at[pl.ds(bl * _TPB, _TPB)], aneg_g)

        pp = [jnp.int32(0)]
        nn = [jnp.int32(0)]
        for t in range(_TPB):
            crow = apos_g[t, pl.ds(_POS_CAP, 16)]
            pp.append(pp[-1] + crow[0])
            nn.append(nn[-1] + crow[1])
        num_pos = jnp.minimum(pp[_TPB], _POS_CAP)
        num_neg = jnp.minimum(nn[_TPB], _NEG_CAP - num_pos)
        tot = num_pos + num_neg

        for chunk in range(_NEG_CAP // 16):
            sv = chunk * 16 + iota16
            is_pos = sv < num_pos
            valid = sv < tot

            tp = jnp.zeros((16,), jnp.int32)
            pv = jnp.zeros((16,), jnp.int32)
            sn = sv - num_pos
            tn = jnp.zeros((16,), jnp.int32)
            nv = jnp.zeros((16,), jnp.int32)
            for t in range(1, _TPB):
                tp = tp + (sv >= pp[t]).astype(jnp.int32)
                tn = tn + (sn >= nn[t]).astype(jnp.int32)
            for t in range(1, _TPB):
                pv = jnp.where(tp == t, pp[t], pv)
                nv = jnp.where(tn == t, nn[t], nv)
            lp = jnp.clip(sv - pv, 0, _POS_CAP - 1)
            ln = jnp.clip(sn - nv, 0, _NEG_CAP - 1)

            gp = plsc.load_gather(apos_g, [tp, lp])
            gn = plsc.load_gather(aneg_g, [tn, ln])
            g = jnp.clip(jnp.where(is_pos, gp, gn), 0, _G - 1)

            lab = jnp.where(is_pos, plsc.load_gather(cls_v, [g]), 0)
            lab_v[pl.ds(chunk * 16, 16)] = lab

            for f in range(4):
                pf = plsc.load_gather(apos_f, [tp + f * _TPB, lp])
                nf = plsc.load_gather(aneg_f, [tn + f * _TPB, ln])
                rois_v[f, pl.ds(chunk * 16, 16)] = jnp.where(
                    valid, jnp.where(is_pos, pf, nf), 0.0)
                gf = plsc.load_gather(
                    gt_v, [jnp.full((16,), f, jnp.int32), g])
                gtb_v[f, pl.ds(chunk * 16, 16)] = jnp.where(valid, gf, 0.0)

        pltpu.sync_copy(rois_v, rois_hbm.at[b])
        pltpu.sync_copy(gtb_v, gtb_hbm.at[b])
        pltpu.sync_copy(lab_v, lab_hbm.at[b])


_sc_kernel = functools.partial(
    pl.kernel,
    compiler_params=pltpu.CompilerParams(needs_layout_passes=False),
    out_type=[
        jax.ShapeDtypeStruct((_B, 4, _NEG_CAP), jnp.float32),
        jax.ShapeDtypeStruct((_B, 4, _NEG_CAP), jnp.float32),
        jax.ShapeDtypeStruct((_B, _NEG_CAP), jnp.int32),
    ],
    mesh=plsc.VectorSubcoreMesh(core_axis_name="c", subcore_axis_name="s",
                                num_cores=2, num_subcores=16),
    scratch_types=[
        pltpu.VMEM((4, _RPAD), jnp.float32),
        pltpu.VMEM((4, _G), jnp.float32),
        pltpu.VMEM((_G,), jnp.int32),
        pltpu.VMEM((_POS_CAP + 16,), jnp.float32),
        pltpu.VMEM((_POS_CAP + 16,), jnp.float32),
        pltpu.VMEM((_POS_CAP + 16,), jnp.float32),
        pltpu.VMEM((_POS_CAP + 16,), jnp.float32),
        pltpu.VMEM((_POS_CAP + 16,), jnp.int32),
        pltpu.VMEM((_NEG_CAP + 16,), jnp.float32),
        pltpu.VMEM((_NEG_CAP + 16,), jnp.float32),
        pltpu.VMEM((_NEG_CAP + 16,), jnp.float32),
        pltpu.VMEM((_NEG_CAP + 16,), jnp.float32),
        pltpu.VMEM((_NEG_CAP + 16,), jnp.int32),
        pltpu.VMEM((4 * _TPB, _POS_CAP), jnp.float32),
        pltpu.VMEM((_TPB, _POS_CAP + 16), jnp.int32),
        pltpu.VMEM((4 * _TPB, _NEG_CAP), jnp.float32),
        pltpu.VMEM((_TPB, _NEG_CAP), jnp.int32),
        pltpu.VMEM((4, _NEG_CAP), jnp.float32),
        pltpu.VMEM((4, _NEG_CAP), jnp.float32),
        pltpu.VMEM((_NEG_CAP,), jnp.int32),
        pltpu.VMEM_SHARED((2 * 2 * 4 * _TPB, _POS_CAP), jnp.float32),
        pltpu.VMEM_SHARED((2 * 2 * _TPB, _POS_CAP + 16), jnp.int32),
        pltpu.VMEM_SHARED((2 * 2 * 4 * _TPB, _NEG_CAP), jnp.float32),
        pltpu.VMEM_SHARED((2 * 2 * _TPB, _NEG_CAP), jnp.int32),
    ],
)(_sc_body)


def _tc_body(rois_ref, gtb_ref, out_rois_ref, out_off_ref):
    x1 = rois_ref[:, 0, :]
    y1 = rois_ref[:, 1, :]
    x2 = rois_ref[:, 2, :]
    y2 = rois_ref[:, 3, :]
    gx1 = gtb_ref[:, 0, :]
    gy1 = gtb_ref[:, 1, :]
    gx2 = gtb_ref[:, 2, :]
    gy2 = gtb_ref[:, 3, :]

    cx = (x1 + x2) * 0.5
    cy = (y1 + y2) * 0.5
    w = x2 - x1
    h = y2 - y1
    gcx = (gx1 + gx2) * 0.5
    gcy = (gy1 + gy2) * 0.5
    gw = gx2 - gx1
    gh = gy2 - gy1

    out_off_ref[:, 0, :] = 10.0 * (gcx - cx) / w
    out_off_ref[:, 1, :] = 10.0 * (gcy - cy) / h
    out_off_ref[:, 2, :] = 5.0 * jnp.log(gw / w)
    out_off_ref[:, 3, :] = 5.0 * jnp.log(gh / h)

    out_rois_ref[:, 0, :] = cx - w * 0.5
    out_rois_ref[:, 1, :] = cy - h * 0.5
    out_rois_ref[:, 2, :] = cx + w * 0.5
    out_rois_ref[:, 3, :] = cy + h * 0.5


_tc_epilogue = pl.pallas_call(
    _tc_body,
    out_shape=[
        jax.ShapeDtypeStruct((_B, 4, _NEG_CAP), jnp.float32),
        jax.ShapeDtypeStruct((_B, 4, _NEG_CAP), jnp.float32),
    ],
)


def kernel(all_proposals, all_gt_bboxes, all_gt_orig_classes):
    props = jnp.concatenate([all_proposals, all_gt_bboxes], axis=1)
    props_t = props.reshape(_B, _TPB, _RPT, 4)
    props_t = jnp.pad(props_t, ((0, 0), (0, 0), (0, _RPAD - _RPT), (0, 0)))
    props_t = props_t.transpose(0, 1, 3, 2)
    gt_t = all_gt_bboxes.transpose(0, 2, 1)
    cls = all_gt_orig_classes.astype(jnp.int32)

    rois_t, gtb_t, labels = _sc_kernel(props_t, gt_t, cls)
    rois_out_t, off_t = _tc_epilogue(rois_t, gtb_t)

    rois_out = rois_out_t.transpose(0, 2, 1)
    gt_offsets = off_t.transpose(0, 2, 1)
    return rois_out, labels, gt_offsets
